# baseline (speedup 1.0000x reference)
import functools

import numpy as np
import jax
import jax.numpy as jnp
from jax import lax
from jax.experimental import pallas as pl
from jax.experimental.pallas import tpu as pltpu
from jax.experimental.pallas import tpu_sc as plsc

_K = 5
_DIM = 3
_NCELL = 64
_NCOR = 8
_BS = 256
_NW = 32
_NSUB = 16
_CH = 128

_BITS = np.array([[(c >> 2) & 1, (c >> 1) & 1, c & 1] for c in range(_NCOR)],
                 dtype=np.int32)

_B0 = np.arange(_NCELL) & 3
_B1 = (np.arange(_NCELL) >> 2) & 3
_B2 = (np.arange(_NCELL) >> 4) & 3
_WIDX = ((_B0[:, None] + _BITS[None, :, 0]) * 1
         + (_B1[:, None] + _BITS[None, :, 1]) * _K
         + (_B2[:, None] + _BITS[None, :, 2]) * _K * _K).astype(np.int32)
_WIDX_FLAT = _WIDX.reshape(-1)


def _sc_gather(h, idx):
    ep = idx.shape[0]
    d = h.shape[1]
    per_w = ep // _NW
    n_ch = per_w // _CH
    mesh = plsc.VectorSubcoreMesh(core_axis_name="c", subcore_axis_name="s")

    @functools.partial(
        pl.kernel, mesh=mesh,
        out_type=jax.ShapeDtypeStruct((ep, d), jnp.float32),
        scratch_types=[
            pltpu.VMEM((_CH,), jnp.int32),
            pltpu.VMEM((_CH, d), jnp.float32),
            pltpu.SemaphoreType.DMA,
        ],
    )
    def k(h_hbm, idx_hbm, out_hbm, idxv, rowsv, sem):
        wid = lax.axis_index("s") * 2 + lax.axis_index("c")
        base = wid * per_w

        def body(i, carry):
            off = base + i * _CH
            pltpu.sync_copy(idx_hbm.at[pl.ds(off, _CH)], idxv)
            pltpu.async_copy(h_hbm.at[idxv], rowsv, sem).wait()
            pltpu.sync_copy(rowsv, out_hbm.at[pl.ds(off, _CH)])
            return carry

        lax.fori_loop(0, n_ch, body, 0)

    return k(h, idx)


def _sc_scatter(msg, dst, zrows):
    ep = msg.shape[0]
    npad, d = zrows.shape
    per_w = ep // _NW
    n_ch = per_w // _CH
    rows_t = npad // _NSUB
    mesh = plsc.VectorSubcoreMesh(core_axis_name="c", subcore_axis_name="s")

    @functools.partial(
        pl.kernel, mesh=mesh,
        out_type=jax.ShapeDtypeStruct((2, npad, d), jnp.float32),
        scratch_types=[
            pltpu.VMEM((_CH,), jnp.int32),
            pltpu.VMEM((_CH, d), jnp.float32),
            pltpu.VMEM_SHARED((npad, d), jnp.float32),
            pltpu.SemaphoreType.DMA,
        ],
    )
    def k(msg_hbm, dst_hbm, z_hbm, out_hbm, idxv, msgv, agg_sh, sem):
        cid = lax.axis_index("c")
        sid = lax.axis_index("s")
        wid = sid * 2 + cid
        pltpu.sync_copy(z_hbm.at[pl.ds(sid * rows_t, rows_t)],
                        agg_sh.at[pl.ds(sid * rows_t, rows_t)])
        plsc.subcore_barrier()

        def body(i, carry):
            off = wid * per_w + i * _CH
            pltpu.sync_copy(dst_hbm.at[pl.ds(off, _CH)], idxv)
            pltpu.sync_copy(msg_hbm.at[pl.ds(off, _CH)], msgv)
            pltpu.sync_copy(msgv, agg_sh.at[idxv], add=True)
            return carry

        lax.fori_loop(0, n_ch, body, 0)
        plsc.subcore_barrier()
        pltpu.sync_copy(agg_sh.at[pl.ds(sid * rows_t, rows_t)],
                        out_hbm.at[cid, pl.ds(sid * rows_t, rows_t)])

    return k(msg, dst, zrows)


def _elu(v):
    return jnp.where(v > 0, v, jnp.exp(jnp.minimum(v, 0.0)) - 1.0)


def _tc_edgemm(cmap, g, b8, wc):
    ep = g.shape[0]
    ci = wc.shape[1] // _NCOR
    co = wc.shape[2]
    nb = ep // _BS

    def body(cmap_ref, g_ref, b_ref, w_ref, o_ref):
        gv = g_ref[...][:, :ci]
        bv = b_ref[...]
        gb = jnp.concatenate([bv[:, c:c + 1] * gv for c in range(_NCOR)],
                             axis=1)
        mm = jnp.dot(gb.astype(jnp.bfloat16), w_ref[0].astype(jnp.bfloat16),
                     preferred_element_type=jnp.float32)
        o_ref[...] = jnp.pad(mm, ((0, 0), (0, 128 - co)))

    return pl.pallas_call(
        body,
        grid_spec=pltpu.PrefetchScalarGridSpec(
            num_scalar_prefetch=1,
            grid=(nb,),
            in_specs=[
                pl.BlockSpec((_BS, 128), lambda j, cm: (j, 0)),
                pl.BlockSpec((_BS, _NCOR), lambda j, cm: (j, 0)),
                pl.BlockSpec((1, _NCOR * ci, co), lambda j, cm: (cm[j], 0, 0)),
            ],
            out_specs=pl.BlockSpec((_BS, 128), lambda j, cm: (j, 0)),
        ),
        out_shape=jax.ShapeDtypeStruct((ep, 128), jnp.float32),
    )(cmap, g, b8, wc)


def _tc_combine(aggp, h, root, bias):
    npad = h.shape[0]
    co = root.shape[1]
    nb = npad // _BS
    bias2 = bias[None, :]

    def body(a_ref, h_ref, r_ref, b_ref, o_ref):
        s = (a_ref[0] + a_ref[1])[:, :co]
        v = s + jnp.dot(h_ref[...], r_ref[...],
                        preferred_element_type=jnp.float32) + b_ref[...]
        o_ref[...] = jnp.pad(_elu(v), ((0, 0), (0, 128 - co)))

    return pl.pallas_call(
        body,
        grid=(nb,),
        in_specs=[
            pl.BlockSpec((2, _BS, 128), lambda j: (0, j, 0)),
            pl.BlockSpec((_BS, 128), lambda j: (j, 0)),
            pl.BlockSpec((128, co), lambda j: (0, 0)),
            pl.BlockSpec((1, co), lambda j: (0, 0)),
        ],
        out_specs=pl.BlockSpec((_BS, 128), lambda j: (j, 0)),
        out_shape=jax.ShapeDtypeStruct((npad, 128), jnp.float32),
    )(aggp, h, root, bias2)


def _tc_tail(h, l1w, l1b, l2w, l2b):
    npad = h.shape[0]
    ci = l1w.shape[0]
    cm = l1w.shape[1]
    cc = l2w.shape[1]
    nb = npad // _BS
    l1b2 = l1b[None, :]
    l2b2 = l2b[None, :]

    def body(h_ref, w1_ref, b1_ref, w2_ref, b2_ref, o_ref):
        a = jnp.dot(h_ref[...][:, :ci], w1_ref[...],
                    preferred_element_type=jnp.float32) + b1_ref[...]
        a = _elu(a)
        z = jnp.dot(a, w2_ref[...],
                    preferred_element_type=jnp.float32) + b2_ref[...]
        m = jnp.max(z, axis=1, keepdims=True)
        lse = m + jnp.log(jnp.sum(jnp.exp(z - m), axis=1, keepdims=True))
        o_ref[...] = z - lse

    return pl.pallas_call(
        body,
        grid=(nb,),
        in_specs=[
            pl.BlockSpec((_BS, 128), lambda j: (j, 0)),
            pl.BlockSpec((ci, cm), lambda j: (0, 0)),
            pl.BlockSpec((1, cm), lambda j: (0, 0)),
            pl.BlockSpec((cm, cc), lambda j: (0, 0)),
            pl.BlockSpec((1, cc), lambda j: (0, 0)),
        ],
        out_specs=pl.BlockSpec((_BS, cc), lambda j: (j, 0)),
        out_shape=jax.ShapeDtypeStruct((npad, cc), jnp.float32),
    )(h, l1w, l1b2, l2w, l2b2)


def kernel(x, edge_index, edge_attr,
           conv1_w, conv1_root, conv1_b, conv2_w, conv2_root, conv2_b,
           conv3_w, conv3_root, conv3_b, conv4_w, conv4_root, conv4_b,
           conv5_w, conv5_root, conv5_b, conv6_w, conv6_root, conv6_b,
           lin1_w, lin1_b, lin2_w, lin2_b):
    n = x.shape[0]
    e = edge_attr.shape[0]
    npad = ((n + _BS - 1) // _BS) * _BS

    nb = e // _BS + _NCELL
    nb = ((nb + 15) // 16) * 16
    ep = nb * _BS

    src = edge_index[0].astype(jnp.int32)
    dst = edge_index[1].astype(jnp.int32)

    v = edge_attr * float(_K - 1)
    botf = jnp.clip(jnp.floor(v), 0.0, float(_K - 2))
    frac = v - botf
    bot = botf.astype(jnp.int32)
    cell = bot[:, 0] + 4 * bot[:, 1] + 16 * bot[:, 2]

    perm = jnp.argsort(cell)
    scell = cell[perm]
    counts = jnp.zeros((_NCELL,), jnp.int32).at[cell].add(1)
    blocks_per = (counts + _BS - 1) // _BS
    pad_start = _BS * (jnp.cumsum(blocks_per) - blocks_per)
    sort_start = jnp.cumsum(counts) - counts
    pos = pad_start[scell] + (jnp.arange(e, dtype=jnp.int32)
                              - sort_start[scell])
    epos = jnp.full((ep,), -1, jnp.int32).at[pos].add(perm + 1)
    valid = epos >= 0
    eidx = jnp.maximum(epos, 0)
    fill = jnp.arange(ep, dtype=jnp.int32)
    srcp = jnp.where(valid, src[eidx], fill % n)
    dstp = jnp.where(valid, dst[eidx], fill % npad)
    fracp = frac[eidx]
    cols = []
    for c in range(_NCOR):
        w = jnp.where(valid, 1.0, 0.0)
        for dim in range(_DIM):
            f = fracp[:, dim]
            w = w * (f if _BITS[c, dim] else 1.0 - f)
        cols.append(w)
    b8p = jnp.stack(cols, axis=1)
    cmap = (jnp.searchsorted(pad_start,
                             jnp.arange(nb, dtype=jnp.int32) * _BS,
                             side="right").astype(jnp.int32) - 1)

    h = jnp.zeros((npad, 128), jnp.float32).at[:n, 0:1].set(x)
    w1p = jnp.pad(conv1_w, ((0, 0), (0, 16 - conv1_w.shape[1]), (0, 0)))

    layers = [
        (w1p, conv1_root, conv1_b),
        (conv2_w, conv2_root, conv2_b),
        (conv3_w, conv3_root, conv3_b),
        (conv4_w, conv4_root, conv4_b),
        (conv5_w, conv5_root, conv5_b),
        (conv6_w, conv6_root, conv6_b),
    ]
    for w, r, b in layers:
        ci, co = w.shape[1], w.shape[2]
        r = jnp.pad(r, ((0, 128 - r.shape[0]), (0, 0)))
        wc = jnp.take(w, _WIDX_FLAT, axis=0).reshape(_NCELL, _NCOR * ci, co)
        g = _sc_gather(h, srcp)
        msg = _tc_edgemm(cmap, g, b8p, wc)
        aggp = _sc_scatter(msg, dstp, jnp.zeros((npad, 128), jnp.float32))
        h = _tc_combine(aggp, h, r, b)

    nclass = lin2_w.shape[1]
    ccpad = ((nclass + 127) // 128) * 128
    l2wp = jnp.pad(lin2_w, ((0, 0), (0, ccpad - nclass)))
    l2bp = jnp.pad(lin2_b, ((0, ccpad - nclass)), constant_values=-1e30)
    out = _tc_tail(h, lin1_w, lin1_b, l2wp, l2bp)
    return out[:n, :nclass]


# speedup vs baseline: 1.1988x; 1.1988x over previous
import functools

import numpy as np
import jax
import jax.numpy as jnp
from jax import lax
from jax.experimental import pallas as pl
from jax.experimental.pallas import tpu as pltpu
from jax.experimental.pallas import tpu_sc as plsc

_K = 5
_DIM = 3
_NCELL = 64
_NCOR = 8
_BS = 256
_NW = 32
_NSUB = 16
_CH = 128

_BITS = np.array([[(c >> 2) & 1, (c >> 1) & 1, c & 1] for c in range(_NCOR)],
                 dtype=np.int32)

_B0 = np.arange(_NCELL) & 3
_B1 = (np.arange(_NCELL) >> 2) & 3
_B2 = (np.arange(_NCELL) >> 4) & 3
_WIDX = ((_B0[:, None] + _BITS[None, :, 0]) * 1
         + (_B1[:, None] + _BITS[None, :, 1]) * _K
         + (_B2[:, None] + _BITS[None, :, 2]) * _K * _K).astype(np.int32)
_WIDX_FLAT = _WIDX.reshape(-1)


def _sc_gather(h, idx):
    ep = idx.shape[0]
    d = h.shape[1]
    per_w = ep // _NW
    n_ch = per_w // _CH
    mesh = plsc.VectorSubcoreMesh(core_axis_name="c", subcore_axis_name="s")

    @functools.partial(
        pl.kernel, mesh=mesh,
        out_type=jax.ShapeDtypeStruct((ep, d), jnp.float32),
        scratch_types=[
            pltpu.VMEM((_CH,), jnp.int32),
            pltpu.VMEM((_CH, d), jnp.float32),
            pltpu.SemaphoreType.DMA,
        ],
    )
    def k(h_hbm, idx_hbm, out_hbm, idxv, rowsv, sem):
        wid = lax.axis_index("s") * 2 + lax.axis_index("c")
        base = wid * per_w

        def body(i, carry):
            off = base + i * _CH
            pltpu.sync_copy(idx_hbm.at[pl.ds(off, _CH)], idxv)
            pltpu.async_copy(h_hbm.at[idxv], rowsv, sem).wait()
            pltpu.sync_copy(rowsv, out_hbm.at[pl.ds(off, _CH)])
            return carry

        lax.fori_loop(0, n_ch, body, 0)

    return k(h, idx)


def _sc_scatter(msg, dst, zrows):
    ep = msg.shape[0]
    npad, d = zrows.shape
    per_w = ep // _NW
    n_ch = per_w // _CH
    rows_t = npad // _NSUB
    mesh = plsc.VectorSubcoreMesh(core_axis_name="c", subcore_axis_name="s")

    @functools.partial(
        pl.kernel, mesh=mesh,
        out_type=jax.ShapeDtypeStruct((2, npad, d), jnp.float32),
        scratch_types=[
            pltpu.VMEM((_CH,), jnp.int32),
            pltpu.VMEM((_CH, d), jnp.float32),
            pltpu.VMEM_SHARED((npad, d), jnp.float32),
            pltpu.SemaphoreType.DMA,
        ],
    )
    def k(msg_hbm, dst_hbm, z_hbm, out_hbm, idxv, msgv, agg_sh, sem):
        cid = lax.axis_index("c")
        sid = lax.axis_index("s")
        wid = sid * 2 + cid
        pltpu.sync_copy(z_hbm.at[pl.ds(sid * rows_t, rows_t)],
                        agg_sh.at[pl.ds(sid * rows_t, rows_t)])
        plsc.subcore_barrier()

        def body(i, carry):
            off = wid * per_w + i * _CH
            pltpu.sync_copy(dst_hbm.at[pl.ds(off, _CH)], idxv)
            pltpu.sync_copy(msg_hbm.at[pl.ds(off, _CH)], msgv)
            pltpu.sync_copy(msgv, agg_sh.at[idxv], add=True)
            return carry

        lax.fori_loop(0, n_ch, body, 0)
        plsc.subcore_barrier()
        pltpu.sync_copy(agg_sh.at[pl.ds(sid * rows_t, rows_t)],
                        out_hbm.at[cid, pl.ds(sid * rows_t, rows_t)])

    return k(msg, dst, zrows)


def _elu(v):
    return jnp.where(v > 0, v, jnp.exp(jnp.minimum(v, 0.0)) - 1.0)


def _tc_edgemm(cmap, g, b8, wh):
    ep = g.shape[0]
    ci = wh.shape[1]
    co = wh.shape[2] // _NCOR
    nb = ep // _BS
    expand = np.zeros((_NCOR, _NCOR * co), np.float32)
    for c in range(_NCOR):
        expand[c, c * co:(c + 1) * co] = 1.0
    expand = jnp.asarray(expand, jnp.bfloat16)

    def body(cmap_ref, g_ref, b_ref, e_ref, w_ref, o_ref):
        gv = g_ref[...][:, :ci].astype(jnp.bfloat16)
        t = jnp.dot(gv, w_ref[0], preferred_element_type=jnp.float32)
        brep = jnp.dot(b_ref[...].astype(jnp.bfloat16), e_ref[...],
                       preferred_element_type=jnp.float32)
        u = brep * t
        width = _NCOR * co
        while width > co:
            width //= 2
            u = u[:, :width] + u[:, width:]
        o_ref[...] = jnp.pad(u, ((0, 0), (0, 128 - co)))

    return pl.pallas_call(
        body,
        grid_spec=pltpu.PrefetchScalarGridSpec(
            num_scalar_prefetch=1,
            grid=(nb,),
            in_specs=[
                pl.BlockSpec((_BS, 128), lambda j, cm: (j, 0)),
                pl.BlockSpec((_BS, _NCOR), lambda j, cm: (j, 0)),
                pl.BlockSpec((_NCOR, _NCOR * co), lambda j, cm: (0, 0)),
                pl.BlockSpec((1, ci, _NCOR * co), lambda j, cm: (cm[j], 0, 0)),
            ],
            out_specs=pl.BlockSpec((_BS, 128), lambda j, cm: (j, 0)),
        ),
        out_shape=jax.ShapeDtypeStruct((ep, 128), jnp.float32),
    )(cmap, g, b8, expand, wh)


def _tc_combine(aggp, h, root, bias):
    npad = h.shape[0]
    co = root.shape[1]
    nb = npad // _BS
    bias2 = bias[None, :]

    def body(a_ref, h_ref, r_ref, b_ref, o_ref):
        s = (a_ref[0] + a_ref[1])[:, :co]
        v = s + jnp.dot(h_ref[...], r_ref[...],
                        preferred_element_type=jnp.float32) + b_ref[...]
        o_ref[...] = jnp.pad(_elu(v), ((0, 0), (0, 128 - co)))

    return pl.pallas_call(
        body,
        grid=(nb,),
        in_specs=[
            pl.BlockSpec((2, _BS, 128), lambda j: (0, j, 0)),
            pl.BlockSpec((_BS, 128), lambda j: (j, 0)),
            pl.BlockSpec((128, co), lambda j: (0, 0)),
            pl.BlockSpec((1, co), lambda j: (0, 0)),
        ],
        out_specs=pl.BlockSpec((_BS, 128), lambda j: (j, 0)),
        out_shape=jax.ShapeDtypeStruct((npad, 128), jnp.float32),
    )(aggp, h, root, bias2)


def _tc_tail(h, l1w, l1b, l2w, l2b):
    npad = h.shape[0]
    ci = l1w.shape[0]
    cm = l1w.shape[1]
    cc = l2w.shape[1]
    nb = npad // _BS
    l1b2 = l1b[None, :]
    l2b2 = l2b[None, :]

    def body(h_ref, w1_ref, b1_ref, w2_ref, b2_ref, o_ref):
        a = jnp.dot(h_ref[...][:, :ci], w1_ref[...],
                    preferred_element_type=jnp.float32) + b1_ref[...]
        a = _elu(a)
        z = jnp.dot(a, w2_ref[...],
                    preferred_element_type=jnp.float32) + b2_ref[...]
        m = jnp.max(z, axis=1, keepdims=True)
        lse = m + jnp.log(jnp.sum(jnp.exp(z - m), axis=1, keepdims=True))
        o_ref[...] = z - lse

    return pl.pallas_call(
        body,
        grid=(nb,),
        in_specs=[
            pl.BlockSpec((_BS, 128), lambda j: (j, 0)),
            pl.BlockSpec((ci, cm), lambda j: (0, 0)),
            pl.BlockSpec((1, cm), lambda j: (0, 0)),
            pl.BlockSpec((cm, cc), lambda j: (0, 0)),
            pl.BlockSpec((1, cc), lambda j: (0, 0)),
        ],
        out_specs=pl.BlockSpec((_BS, cc), lambda j: (j, 0)),
        out_shape=jax.ShapeDtypeStruct((npad, cc), jnp.float32),
    )(h, l1w, l1b2, l2w, l2b2)


def kernel(x, edge_index, edge_attr,
           conv1_w, conv1_root, conv1_b, conv2_w, conv2_root, conv2_b,
           conv3_w, conv3_root, conv3_b, conv4_w, conv4_root, conv4_b,
           conv5_w, conv5_root, conv5_b, conv6_w, conv6_root, conv6_b,
           lin1_w, lin1_b, lin2_w, lin2_b):
    n = x.shape[0]
    e = edge_attr.shape[0]
    npad = ((n + _BS - 1) // _BS) * _BS

    nb = e // _BS + _NCELL
    nb = ((nb + 15) // 16) * 16
    ep = nb * _BS

    src = edge_index[0].astype(jnp.int32)
    dst = edge_index[1].astype(jnp.int32)

    v = edge_attr * float(_K - 1)
    botf = jnp.clip(jnp.floor(v), 0.0, float(_K - 2))
    frac = v - botf
    bot = botf.astype(jnp.int32)
    cell = bot[:, 0] + 4 * bot[:, 1] + 16 * bot[:, 2]
    cols = []
    for c in range(_NCOR):
        w = jnp.ones((e,), jnp.float32)
        for dim in range(_DIM):
            f = frac[:, dim]
            w = w * (f if _BITS[c, dim] else 1.0 - f)
        cols.append(w)
    b8 = jnp.stack(cols, axis=1)

    perm = jnp.argsort(cell)
    scell = cell[perm]
    counts = jnp.zeros((_NCELL,), jnp.int32).at[cell].add(1)
    blocks_per = (counts + _BS - 1) // _BS
    pad_start = _BS * (jnp.cumsum(blocks_per) - blocks_per)
    sort_start = jnp.cumsum(counts) - counts
    pos = pad_start[scell] + (jnp.arange(e, dtype=jnp.int32)
                              - sort_start[scell])
    mask = jnp.zeros((ep,), jnp.int32).at[pos].add(1)
    srcp = jnp.zeros((ep,), jnp.int32).at[pos].add(src[perm])
    dstp = jnp.zeros((ep,), jnp.int32).at[pos].add(dst[perm])
    b8p = jnp.zeros((ep, _NCOR), jnp.float32).at[pos].add(b8[perm])
    fill = jnp.arange(ep, dtype=jnp.int32)
    srcp = jnp.where(mask > 0, srcp, fill % n)
    dstp = jnp.where(mask > 0, dstp, fill % npad)
    cmap = (jnp.searchsorted(pad_start,
                             jnp.arange(nb, dtype=jnp.int32) * _BS,
                             side="right").astype(jnp.int32) - 1)

    h = jnp.zeros((npad, 128), jnp.float32).at[:n, 0:1].set(x)
    w1p = jnp.pad(conv1_w, ((0, 0), (0, 16 - conv1_w.shape[1]), (0, 0)))

    layers = [
        (w1p, conv1_root, conv1_b),
        (conv2_w, conv2_root, conv2_b),
        (conv3_w, conv3_root, conv3_b),
        (conv4_w, conv4_root, conv4_b),
        (conv5_w, conv5_root, conv5_b),
        (conv6_w, conv6_root, conv6_b),
    ]
    for w, r, b in layers:
        ci, co = w.shape[1], w.shape[2]
        r = jnp.pad(r, ((0, 128 - r.shape[0]), (0, 0)))
        wc = (jnp.take(w, _WIDX_FLAT, axis=0)
              .reshape(_NCELL, _NCOR, ci, co)
              .transpose(0, 2, 1, 3)
              .reshape(_NCELL, ci, _NCOR * co)
              .astype(jnp.bfloat16))
        g = _sc_gather(h, srcp)
        msg = _tc_edgemm(cmap, g, b8p, wc)
        aggp = _sc_scatter(msg, dstp, jnp.zeros((npad, 128), jnp.float32))
        h = _tc_combine(aggp, h, r, b)

    nclass = lin2_w.shape[1]
    ccpad = ((nclass + 127) // 128) * 128
    l2wp = jnp.pad(lin2_w, ((0, 0), (0, ccpad - nclass)))
    l2bp = jnp.pad(lin2_b, ((0, ccpad - nclass)), constant_values=-1e30)
    out = _tc_tail(h, lin1_w, lin1_b, l2wp, l2bp)
    return out[:n, :nclass]


# speedup vs baseline: 1.4530x; 1.2120x over previous
import functools

import numpy as np
import jax
import jax.numpy as jnp
from jax import lax
from jax.experimental import pallas as pl
from jax.experimental.pallas import tpu as pltpu
from jax.experimental.pallas import tpu_sc as plsc

_K = 5
_DIM = 3
_NCELL = 64
_NCOR = 8
_BS = 512
_NW = 32
_NSUB = 16
_CH = 128

_BITS = np.array([[(c >> 2) & 1, (c >> 1) & 1, c & 1] for c in range(_NCOR)],
                 dtype=np.int32)

_B0 = np.arange(_NCELL) & 3
_B1 = (np.arange(_NCELL) >> 2) & 3
_B2 = (np.arange(_NCELL) >> 4) & 3
_WIDX = ((_B0[:, None] + _BITS[None, :, 0]) * 1
         + (_B1[:, None] + _BITS[None, :, 1]) * _K
         + (_B2[:, None] + _BITS[None, :, 2]) * _K * _K).astype(np.int32)
_WIDX_FLAT = _WIDX.reshape(-1)


def _sc_gather(h, idx):
    ep = idx.shape[0]
    d = h.shape[1]
    per_w = ep // _NW
    n_ch = per_w // _CH
    mesh = plsc.VectorSubcoreMesh(core_axis_name="c", subcore_axis_name="s")

    @functools.partial(
        pl.kernel, mesh=mesh,
        out_type=jax.ShapeDtypeStruct((ep, d), jnp.float32),
        scratch_types=[
            pltpu.VMEM((_CH,), jnp.int32),
            pltpu.VMEM((_CH, d), jnp.float32),
            pltpu.SemaphoreType.DMA,
        ],
    )
    def k(h_hbm, idx_hbm, out_hbm, idxv, rowsv, sem):
        wid = lax.axis_index("s") * 2 + lax.axis_index("c")
        base = wid * per_w

        def body(i, carry):
            off = base + i * _CH
            pltpu.sync_copy(idx_hbm.at[pl.ds(off, _CH)], idxv)
            pltpu.async_copy(h_hbm.at[idxv], rowsv, sem).wait()
            pltpu.sync_copy(rowsv, out_hbm.at[pl.ds(off, _CH)])
            return carry

        lax.fori_loop(0, n_ch, body, 0)

    return k(h, idx)


def _sc_scatter(msg, dst, zrows):
    ep = msg.shape[0]
    npad, d = zrows.shape
    per_w = ep // _NW
    n_ch = per_w // _CH
    rows_t = npad // _NSUB
    mesh = plsc.VectorSubcoreMesh(core_axis_name="c", subcore_axis_name="s")

    @functools.partial(
        pl.kernel, mesh=mesh,
        out_type=jax.ShapeDtypeStruct((2, npad, d), jnp.float32),
        scratch_types=[
            pltpu.VMEM((_CH,), jnp.int32),
            pltpu.VMEM((_CH, d), jnp.float32),
            pltpu.VMEM_SHARED((npad, d), jnp.float32),
            pltpu.SemaphoreType.DMA,
        ],
    )
    def k(msg_hbm, dst_hbm, z_hbm, out_hbm, idxv, msgv, agg_sh, sem):
        cid = lax.axis_index("c")
        sid = lax.axis_index("s")
        wid = sid * 2 + cid
        pltpu.sync_copy(z_hbm.at[pl.ds(sid * rows_t, rows_t)],
                        agg_sh.at[pl.ds(sid * rows_t, rows_t)])
        plsc.subcore_barrier()

        def body(i, carry):
            off = wid * per_w + i * _CH
            pltpu.sync_copy(dst_hbm.at[pl.ds(off, _CH)], idxv)
            pltpu.sync_copy(msg_hbm.at[pl.ds(off, _CH)], msgv)
            pltpu.sync_copy(msgv, agg_sh.at[idxv], add=True)
            return carry

        lax.fori_loop(0, n_ch, body, 0)
        plsc.subcore_barrier()
        pltpu.sync_copy(agg_sh.at[pl.ds(sid * rows_t, rows_t)],
                        out_hbm.at[cid, pl.ds(sid * rows_t, rows_t)])

    return k(msg, dst, zrows)


def _elu(v):
    return jnp.where(v > 0, v, jnp.exp(jnp.minimum(v, 0.0)) - 1.0)


def _tc_edgemm(cmap, g, b8, wh):
    ep = g.shape[0]
    ci = wh.shape[1]
    co = wh.shape[2] // _NCOR
    nb = ep // _BS
    expand = np.zeros((_NCOR, _NCOR * co), np.float32)
    for c in range(_NCOR):
        expand[c, c * co:(c + 1) * co] = 1.0
    expand = jnp.asarray(expand, jnp.bfloat16)

    def body(cmap_ref, g_ref, b_ref, e_ref, w_ref, o_ref):
        gv = g_ref[...][:, :ci].astype(jnp.bfloat16)
        t = jnp.dot(gv, w_ref[0], preferred_element_type=jnp.float32)
        brep = jnp.dot(b_ref[...].astype(jnp.bfloat16), e_ref[...],
                       preferred_element_type=jnp.float32)
        u = brep * t
        width = _NCOR * co
        while width > co:
            width //= 2
            u = u[:, :width] + u[:, width:]
        o_ref[...] = jnp.pad(u, ((0, 0), (0, 128 - co)))

    return pl.pallas_call(
        body,
        grid_spec=pltpu.PrefetchScalarGridSpec(
            num_scalar_prefetch=1,
            grid=(nb,),
            in_specs=[
                pl.BlockSpec((_BS, 128), lambda j, cm: (j, 0)),
                pl.BlockSpec((_BS, _NCOR), lambda j, cm: (j, 0)),
                pl.BlockSpec((_NCOR, _NCOR * co), lambda j, cm: (0, 0)),
                pl.BlockSpec((1, ci, _NCOR * co), lambda j, cm: (cm[j], 0, 0)),
            ],
            out_specs=pl.BlockSpec((_BS, 128), lambda j, cm: (j, 0)),
        ),
        out_shape=jax.ShapeDtypeStruct((ep, 128), jnp.float32),
    )(cmap, g, b8, expand, wh)


def _tc_combine(aggp, h, root, bias):
    npad = h.shape[0]
    co = root.shape[1]
    nb = npad // _BS
    bias2 = bias[None, :]

    def body(a_ref, h_ref, r_ref, b_ref, o_ref):
        s = (a_ref[0] + a_ref[1])[:, :co]
        v = s + jnp.dot(h_ref[...], r_ref[...],
                        preferred_element_type=jnp.float32) + b_ref[...]
        o_ref[...] = jnp.pad(_elu(v), ((0, 0), (0, 128 - co)))

    return pl.pallas_call(
        body,
        grid=(nb,),
        in_specs=[
            pl.BlockSpec((2, _BS, 128), lambda j: (0, j, 0)),
            pl.BlockSpec((_BS, 128), lambda j: (j, 0)),
            pl.BlockSpec((128, co), lambda j: (0, 0)),
            pl.BlockSpec((1, co), lambda j: (0, 0)),
        ],
        out_specs=pl.BlockSpec((_BS, 128), lambda j: (j, 0)),
        out_shape=jax.ShapeDtypeStruct((npad, 128), jnp.float32),
    )(aggp, h, root, bias2)


def _tc_tail(h, l1w, l1b, l2w, l2b):
    npad = h.shape[0]
    ci = l1w.shape[0]
    cm = l1w.shape[1]
    cc = l2w.shape[1]
    nb = npad // _BS
    l1b2 = l1b[None, :]
    l2b2 = l2b[None, :]

    def body(h_ref, w1_ref, b1_ref, w2_ref, b2_ref, o_ref):
        a = jnp.dot(h_ref[...][:, :ci], w1_ref[...],
                    preferred_element_type=jnp.float32) + b1_ref[...]
        a = _elu(a)
        z = jnp.dot(a, w2_ref[...],
                    preferred_element_type=jnp.float32) + b2_ref[...]
        m = jnp.max(z, axis=1, keepdims=True)
        lse = m + jnp.log(jnp.sum(jnp.exp(z - m), axis=1, keepdims=True))
        o_ref[...] = z - lse

    return pl.pallas_call(
        body,
        grid=(nb,),
        in_specs=[
            pl.BlockSpec((_BS, 128), lambda j: (j, 0)),
            pl.BlockSpec((ci, cm), lambda j: (0, 0)),
            pl.BlockSpec((1, cm), lambda j: (0, 0)),
            pl.BlockSpec((cm, cc), lambda j: (0, 0)),
            pl.BlockSpec((1, cc), lambda j: (0, 0)),
        ],
        out_specs=pl.BlockSpec((_BS, cc), lambda j: (j, 0)),
        out_shape=jax.ShapeDtypeStruct((npad, cc), jnp.float32),
    )(h, l1w, l1b2, l2w, l2b2)


def kernel(x, edge_index, edge_attr,
           conv1_w, conv1_root, conv1_b, conv2_w, conv2_root, conv2_b,
           conv3_w, conv3_root, conv3_b, conv4_w, conv4_root, conv4_b,
           conv5_w, conv5_root, conv5_b, conv6_w, conv6_root, conv6_b,
           lin1_w, lin1_b, lin2_w, lin2_b):
    n = x.shape[0]
    e = edge_attr.shape[0]
    npad = ((n + _BS - 1) // _BS) * _BS

    nb = e // _BS + _NCELL
    nb = ((nb + 15) // 16) * 16
    ep = nb * _BS

    src = edge_index[0].astype(jnp.int32)
    dst = edge_index[1].astype(jnp.int32)

    v = edge_attr * float(_K - 1)
    botf = jnp.clip(jnp.floor(v), 0.0, float(_K - 2))
    frac = v - botf
    bot = botf.astype(jnp.int32)
    cell = bot[:, 0] + 4 * bot[:, 1] + 16 * bot[:, 2]
    cols = []
    for c in range(_NCOR):
        w = jnp.ones((e,), jnp.float32)
        for dim in range(_DIM):
            f = frac[:, dim]
            w = w * (f if _BITS[c, dim] else 1.0 - f)
        cols.append(w)
    b8 = jnp.stack(cols, axis=1)

    perm = jnp.argsort(cell)
    scell = cell[perm]
    counts = jnp.zeros((_NCELL,), jnp.int32).at[cell].add(1)
    blocks_per = (counts + _BS - 1) // _BS
    pad_start = _BS * (jnp.cumsum(blocks_per) - blocks_per)
    sort_start = jnp.cumsum(counts) - counts
    pos = pad_start[scell] + (jnp.arange(e, dtype=jnp.int32)
                              - sort_start[scell])
    packed = jnp.zeros((ep,), jnp.int32).at[pos].add(
        (src[perm] << 15) | (dst[perm] << 1) | 1)
    b8p = jnp.zeros((ep, _NCOR), jnp.float32).at[pos].add(b8[perm])
    mask = packed & 1
    fill = jnp.arange(ep, dtype=jnp.int32)
    srcp = jnp.where(mask > 0, packed >> 15, fill % n)
    dstp = jnp.where(mask > 0, (packed >> 1) & 0x3FFF, fill % npad)
    cmap = (jnp.searchsorted(pad_start,
                             jnp.arange(nb, dtype=jnp.int32) * _BS,
                             side="right").astype(jnp.int32) - 1)

    h = jnp.zeros((npad, 128), jnp.float32).at[:n, 0:1].set(x)
    w1p = jnp.pad(conv1_w, ((0, 0), (0, 16 - conv1_w.shape[1]), (0, 0)))

    layers = [
        (w1p, conv1_root, conv1_b),
        (conv2_w, conv2_root, conv2_b),
        (conv3_w, conv3_root, conv3_b),
        (conv4_w, conv4_root, conv4_b),
        (conv5_w, conv5_root, conv5_b),
        (conv6_w, conv6_root, conv6_b),
    ]
    for w, r, b in layers:
        ci, co = w.shape[1], w.shape[2]
        r = jnp.pad(r, ((0, 128 - r.shape[0]), (0, 0)))
        wc = (jnp.take(w, _WIDX_FLAT, axis=0)
              .reshape(_NCELL, _NCOR, ci, co)
              .transpose(0, 2, 1, 3)
              .reshape(_NCELL, ci, _NCOR * co)
              .astype(jnp.bfloat16))
        g = _sc_gather(h, srcp)
        msg = _tc_edgemm(cmap, g, b8p, wc)
        aggp = _sc_scatter(msg, dstp, jnp.zeros((npad, 128), jnp.float32))
        h = _tc_combine(aggp, h, r, b)

    nclass = lin2_w.shape[1]
    ccpad = ((nclass + 127) // 128) * 128
    l2wp = jnp.pad(lin2_w, ((0, 0), (0, ccpad - nclass)))
    l2bp = jnp.pad(lin2_b, ((0, ccpad - nclass)), constant_values=-1e30)
    out = _tc_tail(h, lin1_w, lin1_b, l2wp, l2bp)
    return out[:n, :nclass]


# speedup vs baseline: 1.5509x; 1.0674x over previous
import functools

import numpy as np
import jax
import jax.numpy as jnp
from jax import lax
from jax.experimental import pallas as pl
from jax.experimental.pallas import tpu as pltpu
from jax.experimental.pallas import tpu_sc as plsc

_K = 5
_DIM = 3
_NCELL = 64
_NCOR = 8
_BS = 512
_NW = 32
_NSUB = 16
_CH = 128

_BITS = np.array([[(c >> 2) & 1, (c >> 1) & 1, c & 1] for c in range(_NCOR)],
                 dtype=np.int32)

_B0 = np.arange(_NCELL) & 3
_B1 = (np.arange(_NCELL) >> 2) & 3
_B2 = (np.arange(_NCELL) >> 4) & 3
_WIDX = ((_B0[:, None] + _BITS[None, :, 0]) * 1
         + (_B1[:, None] + _BITS[None, :, 1]) * _K
         + (_B2[:, None] + _BITS[None, :, 2]) * _K * _K).astype(np.int32)
_WIDX_FLAT = _WIDX.reshape(-1)


def _sc_gather(h, idx):
    ep = idx.shape[0]
    d = h.shape[1]
    per_w = ep // _NW
    n_ch = per_w // _CH
    mesh = plsc.VectorSubcoreMesh(core_axis_name="c", subcore_axis_name="s")

    @functools.partial(
        pl.kernel, mesh=mesh,
        out_type=jax.ShapeDtypeStruct((ep, d), h.dtype),
        scratch_types=[
            pltpu.VMEM((_CH,), jnp.int32),
            pltpu.VMEM((_CH, d), h.dtype),
            pltpu.SemaphoreType.DMA,
        ],
    )
    def k(h_hbm, idx_hbm, out_hbm, idxv, rowsv, sem):
        wid = lax.axis_index("s") * 2 + lax.axis_index("c")
        base = wid * per_w

        def body(i, carry):
            off = base + i * _CH
            pltpu.sync_copy(idx_hbm.at[pl.ds(off, _CH)], idxv)
            pltpu.async_copy(h_hbm.at[idxv], rowsv, sem).wait()
            pltpu.sync_copy(rowsv, out_hbm.at[pl.ds(off, _CH)])
            return carry

        lax.fori_loop(0, n_ch, body, 0)

    return k(h, idx)


def _sc_scatter(msg, dst, zrows):
    ep = msg.shape[0]
    npad, d = zrows.shape
    per_w = ep // _NW
    n_ch = per_w // _CH
    rows_t = npad // _NSUB
    mesh = plsc.VectorSubcoreMesh(core_axis_name="c", subcore_axis_name="s")

    @functools.partial(
        pl.kernel, mesh=mesh,
        out_type=jax.ShapeDtypeStruct((2, npad, d), jnp.float32),
        scratch_types=[
            pltpu.VMEM((_CH,), jnp.int32),
            pltpu.VMEM((_CH, d), jnp.float32),
            pltpu.VMEM_SHARED((npad, d), jnp.float32),
            pltpu.SemaphoreType.DMA,
        ],
    )
    def k(msg_hbm, dst_hbm, z_hbm, out_hbm, idxv, msgv, agg_sh, sem):
        cid = lax.axis_index("c")
        sid = lax.axis_index("s")
        wid = sid * 2 + cid
        pltpu.sync_copy(z_hbm.at[pl.ds(sid * rows_t, rows_t)],
                        agg_sh.at[pl.ds(sid * rows_t, rows_t)])
        plsc.subcore_barrier()

        def body(i, carry):
            off = wid * per_w + i * _CH
            pltpu.sync_copy(dst_hbm.at[pl.ds(off, _CH)], idxv)
            pltpu.sync_copy(msg_hbm.at[pl.ds(off, _CH)], msgv)
            pltpu.sync_copy(msgv, agg_sh.at[idxv], add=True)
            return carry

        lax.fori_loop(0, n_ch, body, 0)
        plsc.subcore_barrier()
        pltpu.sync_copy(agg_sh.at[pl.ds(sid * rows_t, rows_t)],
                        out_hbm.at[cid, pl.ds(sid * rows_t, rows_t)])

    return k(msg, dst, zrows)


def _elu(v):
    return jnp.where(v > 0, v, jnp.exp(jnp.minimum(v, 0.0)) - 1.0)


def _tc_edgemm(cmap, g, b8, wh):
    ep = g.shape[0]
    ci = wh.shape[1]
    co = wh.shape[2] // _NCOR
    nb = ep // _BS
    expand = np.zeros((_NCOR, _NCOR * co), np.float32)
    for c in range(_NCOR):
        expand[c, c * co:(c + 1) * co] = 1.0
    expand = jnp.asarray(expand, jnp.bfloat16)

    def body(cmap_ref, g_ref, b_ref, e_ref, w_ref, o_ref):
        gv = g_ref[...][:, :ci].astype(jnp.bfloat16)
        t = jnp.dot(gv, w_ref[0], preferred_element_type=jnp.float32)
        brep = jnp.dot(b_ref[...].astype(jnp.bfloat16), e_ref[...],
                       preferred_element_type=jnp.float32)
        u = brep * t
        width = _NCOR * co
        while width > co:
            width //= 2
            u = u[:, :width] + u[:, width:]
        o_ref[...] = jnp.pad(u, ((0, 0), (0, 128 - co)))

    return pl.pallas_call(
        body,
        grid_spec=pltpu.PrefetchScalarGridSpec(
            num_scalar_prefetch=1,
            grid=(nb,),
            in_specs=[
                pl.BlockSpec((_BS, 128), lambda j, cm: (j, 0)),
                pl.BlockSpec((_BS, _NCOR), lambda j, cm: (j, 0)),
                pl.BlockSpec((_NCOR, _NCOR * co), lambda j, cm: (0, 0)),
                pl.BlockSpec((1, ci, _NCOR * co), lambda j, cm: (cm[j], 0, 0)),
            ],
            out_specs=pl.BlockSpec((_BS, 128), lambda j, cm: (j, 0)),
        ),
        out_shape=jax.ShapeDtypeStruct((ep, 128), jnp.float32),
    )(cmap, g, b8, expand, wh)


def _tc_combine(aggp, h, root, bias):
    npad = h.shape[0]
    co = root.shape[1]
    nb = npad // _BS
    bias2 = bias[None, :]

    def body(a_ref, h_ref, r_ref, b_ref, o_ref):
        s = (a_ref[0] + a_ref[1])[:, :co]
        v = s + jnp.dot(h_ref[...], r_ref[...],
                        preferred_element_type=jnp.float32) + b_ref[...]
        o_ref[...] = jnp.pad(_elu(v), ((0, 0), (0, 128 - co)))

    return pl.pallas_call(
        body,
        grid=(nb,),
        in_specs=[
            pl.BlockSpec((2, _BS, 128), lambda j: (0, j, 0)),
            pl.BlockSpec((_BS, 128), lambda j: (j, 0)),
            pl.BlockSpec((128, co), lambda j: (0, 0)),
            pl.BlockSpec((1, co), lambda j: (0, 0)),
        ],
        out_specs=pl.BlockSpec((_BS, 128), lambda j: (j, 0)),
        out_shape=jax.ShapeDtypeStruct((npad, 128), jnp.float32),
    )(aggp, h, root, bias2)


def _tc_tail(h, l1w, l1b, l2w, l2b):
    npad = h.shape[0]
    ci = l1w.shape[0]
    cm = l1w.shape[1]
    cc = l2w.shape[1]
    nb = npad // _BS
    l1b2 = l1b[None, :]
    l2b2 = l2b[None, :]

    def body(h_ref, w1_ref, b1_ref, w2_ref, b2_ref, o_ref):
        a = jnp.dot(h_ref[...][:, :ci], w1_ref[...],
                    preferred_element_type=jnp.float32) + b1_ref[...]
        a = _elu(a)
        z = jnp.dot(a, w2_ref[...],
                    preferred_element_type=jnp.float32) + b2_ref[...]
        m = jnp.max(z, axis=1, keepdims=True)
        lse = m + jnp.log(jnp.sum(jnp.exp(z - m), axis=1, keepdims=True))
        o_ref[...] = z - lse

    return pl.pallas_call(
        body,
        grid=(nb,),
        in_specs=[
            pl.BlockSpec((_BS, 128), lambda j: (j, 0)),
            pl.BlockSpec((ci, cm), lambda j: (0, 0)),
            pl.BlockSpec((1, cm), lambda j: (0, 0)),
            pl.BlockSpec((cm, cc), lambda j: (0, 0)),
            pl.BlockSpec((1, cc), lambda j: (0, 0)),
        ],
        out_specs=pl.BlockSpec((_BS, cc), lambda j: (j, 0)),
        out_shape=jax.ShapeDtypeStruct((npad, cc), jnp.float32),
    )(h, l1w, l1b2, l2w, l2b2)


def kernel(x, edge_index, edge_attr,
           conv1_w, conv1_root, conv1_b, conv2_w, conv2_root, conv2_b,
           conv3_w, conv3_root, conv3_b, conv4_w, conv4_root, conv4_b,
           conv5_w, conv5_root, conv5_b, conv6_w, conv6_root, conv6_b,
           lin1_w, lin1_b, lin2_w, lin2_b):
    n = x.shape[0]
    e = edge_attr.shape[0]
    npad = ((n + _BS - 1) // _BS) * _BS

    nb = e // _BS + _NCELL
    nb = ((nb + 15) // 16) * 16
    ep = nb * _BS

    src = edge_index[0].astype(jnp.int32)
    dst = edge_index[1].astype(jnp.int32)

    v = edge_attr * float(_K - 1)
    botf = jnp.clip(jnp.floor(v), 0.0, float(_K - 2))
    frac = v - botf
    bot = botf.astype(jnp.int32)
    cell = bot[:, 0] + 4 * bot[:, 1] + 16 * bot[:, 2]

    perm = jnp.argsort(cell)
    scell = cell[perm]
    counts = jnp.zeros((_NCELL,), jnp.int32).at[cell].add(1)
    blocks_per = (counts + _BS - 1) // _BS
    pad_start = _BS * (jnp.cumsum(blocks_per) - blocks_per)
    sort_start = jnp.cumsum(counts) - counts
    pos = pad_start[scell] + (jnp.arange(e, dtype=jnp.int32)
                              - sort_start[scell])
    packed = jnp.zeros((ep,), jnp.int32).at[pos].add(
        (src[perm] << 15) | (dst[perm] << 1) | 1)
    fq = jnp.round(frac * 1023.0).astype(jnp.int32)
    packedf = jnp.zeros((ep,), jnp.int32).at[pos].add(
        ((fq[:, 0] << 20) | (fq[:, 1] << 10) | fq[:, 2])[perm])
    mask = packed & 1
    fill = jnp.arange(ep, dtype=jnp.int32)
    srcp = jnp.where(mask > 0, packed >> 15, fill % n)
    dstp = jnp.where(mask > 0, (packed >> 1) & 0x3FFF, fill % npad)
    fracp = jnp.stack(
        [((packedf >> 20) & 1023).astype(jnp.float32) * (1.0 / 1023.0),
         ((packedf >> 10) & 1023).astype(jnp.float32) * (1.0 / 1023.0),
         (packedf & 1023).astype(jnp.float32) * (1.0 / 1023.0)], axis=1)
    cols = []
    for c in range(_NCOR):
        w = mask.astype(jnp.float32)
        for dim in range(_DIM):
            f = fracp[:, dim]
            w = w * (f if _BITS[c, dim] else 1.0 - f)
        cols.append(w)
    b8p = jnp.stack(cols, axis=1)
    cmap = (jnp.searchsorted(pad_start,
                             jnp.arange(nb, dtype=jnp.int32) * _BS,
                             side="right").astype(jnp.int32) - 1)

    h = jnp.zeros((npad, 128), jnp.float32).at[:n, 0:1].set(x)
    w1p = jnp.pad(conv1_w, ((0, 0), (0, 16 - conv1_w.shape[1]), (0, 0)))

    layers = [
        (w1p, conv1_root, conv1_b),
        (conv2_w, conv2_root, conv2_b),
        (conv3_w, conv3_root, conv3_b),
        (conv4_w, conv4_root, conv4_b),
        (conv5_w, conv5_root, conv5_b),
        (conv6_w, conv6_root, conv6_b),
    ]
    for w, r, b in layers:
        ci, co = w.shape[1], w.shape[2]
        r = jnp.pad(r, ((0, 128 - r.shape[0]), (0, 0)))
        wc = (jnp.take(w, _WIDX_FLAT, axis=0)
              .reshape(_NCELL, _NCOR, ci, co)
              .transpose(0, 2, 1, 3)
              .reshape(_NCELL, ci, _NCOR * co)
              .astype(jnp.bfloat16))
        g = _sc_gather(h, srcp)
        msg = _tc_edgemm(cmap, g, b8p, wc)
        aggp = _sc_scatter(msg, dstp, jnp.zeros((npad, 128), jnp.float32))
        h = _tc_combine(aggp, h, r, b)

    nclass = lin2_w.shape[1]
    ccpad = ((nclass + 127) // 128) * 128
    l2wp = jnp.pad(lin2_w, ((0, 0), (0, ccpad - nclass)))
    l2bp = jnp.pad(lin2_b, ((0, ccpad - nclass)), constant_values=-1e30)
    out = _tc_tail(h, lin1_w, lin1_b, l2wp, l2bp)
    return out[:n, :nclass]


# speedup vs baseline: 1.6185x; 1.0436x over previous
import functools

import numpy as np
import jax
import jax.numpy as jnp
from jax import lax
from jax.experimental import pallas as pl
from jax.experimental.pallas import tpu as pltpu
from jax.experimental.pallas import tpu_sc as plsc

_K = 5
_DIM = 3
_NCELL = 64
_NCOR = 8
_BS = 512
_NW = 32
_NSUB = 16
_CH = 128

_BITS = np.array([[(c >> 2) & 1, (c >> 1) & 1, c & 1] for c in range(_NCOR)],
                 dtype=np.int32)

_B0 = np.arange(_NCELL) & 3
_B1 = (np.arange(_NCELL) >> 2) & 3
_B2 = (np.arange(_NCELL) >> 4) & 3
_WIDX = ((_B0[:, None] + _BITS[None, :, 0]) * 1
         + (_B1[:, None] + _BITS[None, :, 1]) * _K
         + (_B2[:, None] + _BITS[None, :, 2]) * _K * _K).astype(np.int32)
_WIDX_FLAT = _WIDX.reshape(-1)


def _sc_gather(h, idx):
    ep = idx.shape[0]
    d = h.shape[1]
    per_w = ep // _NW
    n_it = per_w // (4 * _CH)
    idx2 = idx.reshape(ep // _CH, _CH)
    mesh = plsc.VectorSubcoreMesh(core_axis_name="c", subcore_axis_name="s")

    @functools.partial(
        pl.kernel, mesh=mesh,
        out_type=jax.ShapeDtypeStruct((ep, d), h.dtype),
        scratch_types=[
            pltpu.VMEM((4, _CH), jnp.int32),
            pltpu.VMEM((4 * _CH, d), h.dtype),
            pltpu.SemaphoreType.DMA,
        ],
    )
    def k(h_hbm, idx_hbm, out_hbm, idxv, rowsv, sem):
        wid = lax.axis_index("s") * 2 + lax.axis_index("c")
        base_u = wid * (per_w // _CH)

        def body(i, carry):
            u = base_u + i * 4
            pltpu.sync_copy(idx_hbm.at[pl.ds(u, 4)], idxv)
            cps = [pltpu.async_copy(h_hbm.at[idxv.at[q]],
                                    rowsv.at[pl.ds(q * _CH, _CH)], sem)
                   for q in range(4)]
            for cp in cps:
                cp.wait()
            pltpu.sync_copy(rowsv, out_hbm.at[pl.ds(u * _CH, 4 * _CH)])
            return carry

        lax.fori_loop(0, n_it, body, 0)

    return k(h, idx2)


def _sc_scatter(msg, dst, zrows):
    ep = msg.shape[0]
    npad, d = zrows.shape
    per_w = ep // _NW
    n_ch = per_w // _CH
    rows_t = npad // _NSUB
    mesh = plsc.VectorSubcoreMesh(core_axis_name="c", subcore_axis_name="s")

    @functools.partial(
        pl.kernel, mesh=mesh,
        out_type=jax.ShapeDtypeStruct((2, npad, d), jnp.float32),
        scratch_types=[
            pltpu.VMEM((_CH,), jnp.int32),
            pltpu.VMEM((_CH, d), jnp.float32),
            pltpu.VMEM_SHARED((npad, d), jnp.float32),
            pltpu.SemaphoreType.DMA,
        ],
    )
    def k(msg_hbm, dst_hbm, z_hbm, out_hbm, idxv, msgv, agg_sh, sem):
        cid = lax.axis_index("c")
        sid = lax.axis_index("s")
        wid = sid * 2 + cid
        pltpu.sync_copy(z_hbm.at[pl.ds(sid * rows_t, rows_t)],
                        agg_sh.at[pl.ds(sid * rows_t, rows_t)])
        plsc.subcore_barrier()

        def body(i, carry):
            off = wid * per_w + i * _CH
            pltpu.sync_copy(dst_hbm.at[pl.ds(off, _CH)], idxv)
            pltpu.sync_copy(msg_hbm.at[pl.ds(off, _CH)], msgv)
            pltpu.sync_copy(msgv, agg_sh.at[idxv], add=True)
            return carry

        lax.fori_loop(0, n_ch, body, 0)
        plsc.subcore_barrier()
        pltpu.sync_copy(agg_sh.at[pl.ds(sid * rows_t, rows_t)],
                        out_hbm.at[cid, pl.ds(sid * rows_t, rows_t)])

    return k(msg, dst, zrows)


def _elu(v):
    return jnp.where(v > 0, v, jnp.exp(jnp.minimum(v, 0.0)) - 1.0)


def _tc_edgemm(cmap, g, b8, wh):
    ep = g.shape[0]
    ci = wh.shape[1]
    co = wh.shape[2] // _NCOR
    nb = ep // _BS
    expand = np.zeros((_NCOR, _NCOR * co), np.float32)
    for c in range(_NCOR):
        expand[c, c * co:(c + 1) * co] = 1.0
    expand = jnp.asarray(expand, jnp.bfloat16)

    def body(cmap_ref, g_ref, b_ref, e_ref, w_ref, o_ref):
        gv = g_ref[...][:, :ci].astype(jnp.bfloat16)
        t = jnp.dot(gv, w_ref[0], preferred_element_type=jnp.float32)
        brep = jnp.dot(b_ref[...].astype(jnp.bfloat16), e_ref[...],
                       preferred_element_type=jnp.float32)
        u = brep * t
        width = _NCOR * co
        while width > co:
            width //= 2
            u = u[:, :width] + u[:, width:]
        o_ref[...] = jnp.pad(u, ((0, 0), (0, 128 - co)))

    return pl.pallas_call(
        body,
        grid_spec=pltpu.PrefetchScalarGridSpec(
            num_scalar_prefetch=1,
            grid=(nb,),
            in_specs=[
                pl.BlockSpec((_BS, 128), lambda j, cm: (j, 0)),
                pl.BlockSpec((_BS, _NCOR), lambda j, cm: (j, 0)),
                pl.BlockSpec((_NCOR, _NCOR * co), lambda j, cm: (0, 0)),
                pl.BlockSpec((1, ci, _NCOR * co), lambda j, cm: (cm[j], 0, 0)),
            ],
            out_specs=pl.BlockSpec((_BS, 128), lambda j, cm: (j, 0)),
        ),
        out_shape=jax.ShapeDtypeStruct((ep, 128), jnp.float32),
    )(cmap, g, b8, expand, wh)


def _tc_combine(aggp, h, root, bias):
    npad = h.shape[0]
    co = root.shape[1]
    nb = npad // _BS
    bias2 = bias[None, :]

    def body(a_ref, h_ref, r_ref, b_ref, o_ref):
        s = (a_ref[0] + a_ref[1])[:, :co]
        v = s + jnp.dot(h_ref[...], r_ref[...],
                        preferred_element_type=jnp.float32) + b_ref[...]
        o_ref[...] = jnp.pad(_elu(v), ((0, 0), (0, 128 - co)))

    return pl.pallas_call(
        body,
        grid=(nb,),
        in_specs=[
            pl.BlockSpec((2, _BS, 128), lambda j: (0, j, 0)),
            pl.BlockSpec((_BS, 128), lambda j: (j, 0)),
            pl.BlockSpec((128, co), lambda j: (0, 0)),
            pl.BlockSpec((1, co), lambda j: (0, 0)),
        ],
        out_specs=pl.BlockSpec((_BS, 128), lambda j: (j, 0)),
        out_shape=jax.ShapeDtypeStruct((npad, 128), jnp.float32),
    )(aggp, h, root, bias2)


def _tc_tail(h, l1w, l1b, l2w, l2b):
    npad = h.shape[0]
    ci = l1w.shape[0]
    cm = l1w.shape[1]
    cc = l2w.shape[1]
    nb = npad // _BS
    l1b2 = l1b[None, :]
    l2b2 = l2b[None, :]

    def body(h_ref, w1_ref, b1_ref, w2_ref, b2_ref, o_ref):
        a = jnp.dot(h_ref[...][:, :ci], w1_ref[...],
                    preferred_element_type=jnp.float32) + b1_ref[...]
        a = _elu(a)
        z = jnp.dot(a, w2_ref[...],
                    preferred_element_type=jnp.float32) + b2_ref[...]
        m = jnp.max(z, axis=1, keepdims=True)
        lse = m + jnp.log(jnp.sum(jnp.exp(z - m), axis=1, keepdims=True))
        o_ref[...] = z - lse

    return pl.pallas_call(
        body,
        grid=(nb,),
        in_specs=[
            pl.BlockSpec((_BS, 128), lambda j: (j, 0)),
            pl.BlockSpec((ci, cm), lambda j: (0, 0)),
            pl.BlockSpec((1, cm), lambda j: (0, 0)),
            pl.BlockSpec((cm, cc), lambda j: (0, 0)),
            pl.BlockSpec((1, cc), lambda j: (0, 0)),
        ],
        out_specs=pl.BlockSpec((_BS, cc), lambda j: (j, 0)),
        out_shape=jax.ShapeDtypeStruct((npad, cc), jnp.float32),
    )(h, l1w, l1b2, l2w, l2b2)


def kernel(x, edge_index, edge_attr,
           conv1_w, conv1_root, conv1_b, conv2_w, conv2_root, conv2_b,
           conv3_w, conv3_root, conv3_b, conv4_w, conv4_root, conv4_b,
           conv5_w, conv5_root, conv5_b, conv6_w, conv6_root, conv6_b,
           lin1_w, lin1_b, lin2_w, lin2_b):
    n = x.shape[0]
    e = edge_attr.shape[0]
    npad = ((n + _BS - 1) // _BS) * _BS

    nb = e // _BS + _NCELL
    nb = ((nb + 15) // 16) * 16
    ep = nb * _BS

    src = edge_index[0].astype(jnp.int32)
    dst = edge_index[1].astype(jnp.int32)

    v = edge_attr * float(_K - 1)
    botf = jnp.clip(jnp.floor(v), 0.0, float(_K - 2))
    frac = v - botf
    bot = botf.astype(jnp.int32)
    cell = bot[:, 0] + 4 * bot[:, 1] + 16 * bot[:, 2]

    perm = jnp.argsort(cell)
    scell = cell[perm]
    counts = jnp.zeros((_NCELL,), jnp.int32).at[cell].add(1)
    blocks_per = (counts + _BS - 1) // _BS
    pad_start = _BS * (jnp.cumsum(blocks_per) - blocks_per)
    sort_start = jnp.cumsum(counts) - counts
    pos = pad_start[scell] + (jnp.arange(e, dtype=jnp.int32)
                              - sort_start[scell])
    packed = jnp.zeros((ep,), jnp.int32).at[pos].add(
        (src[perm] << 15) | (dst[perm] << 1) | 1)
    fq = jnp.round(frac * 1023.0).astype(jnp.int32)
    packedf = jnp.zeros((ep,), jnp.int32).at[pos].add(
        ((fq[:, 0] << 20) | (fq[:, 1] << 10) | fq[:, 2])[perm])
    mask = packed & 1
    fill = jnp.arange(ep, dtype=jnp.int32)
    srcp = jnp.where(mask > 0, packed >> 15, fill % n)
    dstp = jnp.where(mask > 0, (packed >> 1) & 0x3FFF, fill % npad)
    fracp = jnp.stack(
        [((packedf >> 20) & 1023).astype(jnp.float32) * (1.0 / 1023.0),
         ((packedf >> 10) & 1023).astype(jnp.float32) * (1.0 / 1023.0),
         (packedf & 1023).astype(jnp.float32) * (1.0 / 1023.0)], axis=1)
    cols = []
    for c in range(_NCOR):
        w = mask.astype(jnp.float32)
        for dim in range(_DIM):
            f = fracp[:, dim]
            w = w * (f if _BITS[c, dim] else 1.0 - f)
        cols.append(w)
    b8p = jnp.stack(cols, axis=1)
    cmap = (jnp.searchsorted(pad_start,
                             jnp.arange(nb, dtype=jnp.int32) * _BS,
                             side="right").astype(jnp.int32) - 1)

    h = jnp.zeros((npad, 128), jnp.float32).at[:n, 0:1].set(x)
    w1p = jnp.pad(conv1_w, ((0, 0), (0, 16 - conv1_w.shape[1]), (0, 0)))

    layers = [
        (w1p, conv1_root, conv1_b),
        (conv2_w, conv2_root, conv2_b),
        (conv3_w, conv3_root, conv3_b),
        (conv4_w, conv4_root, conv4_b),
        (conv5_w, conv5_root, conv5_b),
        (conv6_w, conv6_root, conv6_b),
    ]
    for w, r, b in layers:
        ci, co = w.shape[1], w.shape[2]
        r = jnp.pad(r, ((0, 128 - r.shape[0]), (0, 0)))
        wc = (jnp.take(w, _WIDX_FLAT, axis=0)
              .reshape(_NCELL, _NCOR, ci, co)
              .transpose(0, 2, 1, 3)
              .reshape(_NCELL, ci, _NCOR * co)
              .astype(jnp.bfloat16))
        g = _sc_gather(h, srcp)
        msg = _tc_edgemm(cmap, g, b8p, wc)
        aggp = _sc_scatter(msg, dstp, jnp.zeros((npad, 128), jnp.float32))
        h = _tc_combine(aggp, h, r, b)

    nclass = lin2_w.shape[1]
    ccpad = ((nclass + 127) // 128) * 128
    l2wp = jnp.pad(lin2_w, ((0, 0), (0, ccpad - nclass)))
    l2bp = jnp.pad(lin2_b, ((0, ccpad - nclass)), constant_values=-1e30)
    out = _tc_tail(h, lin1_w, lin1_b, l2wp, l2bp)
    return out[:n, :nclass]
